# baseline (speedup 1.0000x reference)
import functools

import jax
import jax.numpy as jnp
from jax import lax
from jax.experimental import pallas as pl
from jax.experimental.pallas import tpu as pltpu
from jax.experimental.pallas import tpu_sc as plsc

N = 10000
E = 320000
D = 128
NC = 2
NS = 16
K = 80
STRIDE = 624
WPT = 640
NCH = WPT // K

_mesh = plsc.VectorSubcoreMesh(core_axis_name="c", subcore_axis_name="s")

_f32 = jnp.float32


def _gather_scatter_loop(nchunks, ebase, src_off, xref, src_hbm, dst_hbm,
                         s_sh, bufs, semI, semG, semS):
    G = nchunks

    def idx_start(i, g):
        e0 = ebase + g * K
        pltpu.async_copy(src_hbm.at[pl.ds(src_off + e0, K)], bufs[i][0],
                         semI[i][0])
        pltpu.async_copy(dst_hbm.at[pl.ds(e0, K)], bufs[i][1], semI[i][1])

    def idx_wait(i):
        pltpu.make_async_copy(src_hbm.at[pl.ds(0, K)], bufs[i][0],
                              semI[i][0]).wait()
        pltpu.make_async_copy(dst_hbm.at[pl.ds(0, K)], bufs[i][1],
                              semI[i][1]).wait()

    def g_start(i):
        pltpu.async_copy(xref.at[bufs[i][0]], bufs[i][2], semG[i])

    def g_wait(i):
        pltpu.make_async_copy(xref.at[bufs[i][0]], bufs[i][2], semG[i]).wait()

    def s_start(i):
        pltpu.async_copy(bufs[i][2], s_sh.at[bufs[i][1]], semS[i], add=True)

    def s_wait(i):
        pltpu.make_async_copy(bufs[i][2], s_sh.at[bufs[i][1]], semS[i]).wait()

    def sub(i0, t):
        idx_wait((i0 + 2) % 3)
        g_start((i0 + 2) % 3)
        g_wait((i0 + 1) % 3)
        s_start((i0 + 1) % 3)
        s_wait(i0)
        idx_start(i0, t)

    idx_start(0, 0)
    idx_start(1, 1)
    idx_wait(0)
    g_start(0)
    idx_start(2, 2)
    idx_wait(1)
    g_start(1)
    g_wait(0)
    s_start(0)

    nloop = (G - 3) // 3

    def body(g2, carry):
        t = 3 + 3 * g2
        sub(0, t)
        sub(1, t + 1)
        sub(2, t + 2)
        return carry
    lax.fori_loop(0, nloop, body, 0)

    for t in range(3 + 3 * nloop, G):
        sub(t % 3, t)
    i = G % 3
    idx_wait((i + 2) % 3)
    g_start((i + 2) % 3)
    g_wait((i + 1) % 3)
    s_start((i + 1) % 3)
    s_wait(i)
    i = (G + 1) % 3
    g_wait((i + 1) % 3)
    s_start((i + 1) % 3)
    s_wait(i)
    s_wait((G - 1) % 3)


@functools.partial(
    pl.kernel,
    out_type=jax.ShapeDtypeStruct((2 * N, D), _f32),
    mesh=_mesh,
    scratch_types=(
        pltpu.VMEM((K,), jnp.int32),
        pltpu.VMEM((K,), jnp.int32),
        pltpu.VMEM((K, D), _f32),
        pltpu.VMEM((K,), jnp.int32),
        pltpu.VMEM((K,), jnp.int32),
        pltpu.VMEM((K, D), _f32),
        pltpu.VMEM((K,), jnp.int32),
        pltpu.VMEM((K,), jnp.int32),
        pltpu.VMEM((K, D), _f32),
        pltpu.VMEM_SHARED((N, D), _f32),
        pltpu.SemaphoreType.DMA, pltpu.SemaphoreType.DMA,
        pltpu.SemaphoreType.DMA, pltpu.SemaphoreType.DMA,
        pltpu.SemaphoreType.DMA, pltpu.SemaphoreType.DMA,
        pltpu.SemaphoreType.DMA, pltpu.SemaphoreType.DMA,
        pltpu.SemaphoreType.DMA,
        pltpu.SemaphoreType.DMA, pltpu.SemaphoreType.DMA,
        pltpu.SemaphoreType.DMA,
    ),
)
def _agg1(xh2_hbm, src2_hbm, dst_hbm, zrow_hbm,
          s01_out,
          s0, d0, r0_, s1, d1, r1, s2, d2, r2, s_sh,
          i0a, i0b, i1a, i1b, i2a, i2b, g0, g1, g2, t0, t1, t2):
    cid = lax.axis_index("c")
    sid = lax.axis_index("s")
    ept = E // NS
    rbase = sid * STRIDE

    pltpu.sync_copy(zrow_hbm, r0_)
    for t in range(NCH):
        pltpu.sync_copy(r0_, s_sh.at[pl.ds(rbase + t * K, K)])
    plsc.subcore_barrier()

    _gather_scatter_loop(ept // K, sid * ept, cid * E, xh2_hbm,
                         src2_hbm, dst_hbm, s_sh,
                         ((s0, d0, r0_), (s1, d1, r1), (s2, d2, r2)),
                         ((i0a, i0b), (i1a, i1b), (i2a, i2b)),
                         (g0, g1, g2), (t0, t1, t2))

    plsc.subcore_barrier()

    for t in range(NCH):
        r0 = rbase + t * K
        pltpu.sync_copy(s_sh.at[pl.ds(r0, K)], r0_)
        pltpu.sync_copy(r0_, s01_out.at[pl.ds(cid * N + r0, K)])


@functools.partial(
    pl.kernel,
    out_type=jax.ShapeDtypeStruct((2 * N, D), _f32),
    mesh=_mesh,
    scratch_types=(
        pltpu.VMEM((K,), jnp.int32),
        pltpu.VMEM((K,), jnp.int32),
        pltpu.VMEM((K, D), _f32),
        pltpu.VMEM_SHARED((N, D), _f32),
        pltpu.SemaphoreType.DMA,
        pltpu.SemaphoreType.DMA,
    ),
)
def _degk(dst_hbm, zrow_hbm, ones_hbm,
          dg_out,
          dstA, dstB, ones_v, d_sh, semA, semB):
    cid = lax.axis_index("c")
    sid = lax.axis_index("s")
    epw = E // (NC * NS)
    rbase = sid * STRIDE
    ebase = (cid * NS + sid) * epw
    nchunks = epw // K

    pltpu.sync_copy(zrow_hbm, ones_v)
    for t in range(NCH):
        pltpu.sync_copy(ones_v, d_sh.at[pl.ds(rbase + t * K, K)])
    pltpu.sync_copy(ones_hbm, ones_v)
    plsc.subcore_barrier()

    def issue(buf_dst, sem, g):
        pltpu.sync_copy(dst_hbm.at[pl.ds(ebase + g * K, K)], buf_dst)
        pltpu.async_copy(ones_v, d_sh.at[buf_dst], sem, add=True)

    def drain(buf_dst, sem):
        pltpu.make_async_copy(ones_v, d_sh.at[buf_dst], sem).wait()

    issue(dstA, semA, 0)
    nloop = (nchunks - 2) // 2

    def body(g2, carry):
        issue(dstB, semB, 2 * g2 + 1)
        drain(dstA, semA)
        issue(dstA, semA, 2 * g2 + 2)
        drain(dstB, semB)
        return carry
    lax.fori_loop(0, nloop, body, 0)

    c0 = 2 * nloop
    if nchunks % 2 == 0:
        issue(dstB, semB, c0 + 1)
        drain(dstA, semA)
        drain(dstB, semB)
    else:
        issue(dstB, semB, c0 + 1)
        drain(dstA, semA)
        issue(dstA, semA, c0 + 2)
        drain(dstB, semB)
        drain(dstA, semA)

    plsc.subcore_barrier()

    for t in range(NCH):
        r0 = rbase + t * K
        pltpu.sync_copy(d_sh.at[pl.ds(r0, K)], ones_v)
        pltpu.sync_copy(ones_v, dg_out.at[pl.ds(cid * N + r0, K)])


@functools.partial(
    pl.kernel,
    out_type=jax.ShapeDtypeStruct((2 * N, D), _f32),
    mesh=_mesh,
    scratch_types=(
        pltpu.VMEM((K,), jnp.int32),
        pltpu.VMEM((K,), jnp.int32),
        pltpu.VMEM((K, D), _f32),
        pltpu.VMEM((K,), jnp.int32),
        pltpu.VMEM((K,), jnp.int32),
        pltpu.VMEM((K, D), _f32),
        pltpu.VMEM((K,), jnp.int32),
        pltpu.VMEM((K,), jnp.int32),
        pltpu.VMEM((K, D), _f32),
        pltpu.VMEM_SHARED((N, D), _f32),
        pltpu.SemaphoreType.DMA, pltpu.SemaphoreType.DMA,
        pltpu.SemaphoreType.DMA, pltpu.SemaphoreType.DMA,
        pltpu.SemaphoreType.DMA, pltpu.SemaphoreType.DMA,
        pltpu.SemaphoreType.DMA, pltpu.SemaphoreType.DMA,
        pltpu.SemaphoreType.DMA,
        pltpu.SemaphoreType.DMA, pltpu.SemaphoreType.DMA,
        pltpu.SemaphoreType.DMA,
    ),
)
def _agg2(rh_hbm, src_hbm, dst_hbm, zrow_hbm,
          s2_out,
          s0, d0, r0_, s1, d1, r1, s2, d2, r2, s_sh,
          i0a, i0b, i1a, i1b, i2a, i2b, g0, g1, g2, t0, t1, t2):
    cid = lax.axis_index("c")
    sid = lax.axis_index("s")
    epw = E // (NC * NS)
    rbase = sid * STRIDE

    pltpu.sync_copy(zrow_hbm, r0_)
    for t in range(NCH):
        pltpu.sync_copy(r0_, s_sh.at[pl.ds(rbase + t * K, K)])
    plsc.subcore_barrier()

    wid = cid * NS + sid
    _gather_scatter_loop(epw // K, wid * epw, 0, rh_hbm,
                         src_hbm, dst_hbm, s_sh,
                         ((s0, d0, r0_), (s1, d1, r1), (s2, d2, r2)),
                         ((i0a, i0b), (i1a, i1b), (i2a, i2b)),
                         (g0, g1, g2), (t0, t1, t2))

    plsc.subcore_barrier()

    for t in range(NCH):
        r0 = rbase + t * K
        pltpu.sync_copy(s_sh.at[pl.ds(r0, K)], r0_)
        pltpu.sync_copy(r0_, s2_out.at[pl.ds(cid * N + r0, K)])


BLK = 2000
NBLK = N // BLK


def _gates_body(s0a, s1a, dga, dgb, hp, wz, bz, wr, br, z_out, rh_out):
    d = dga[...][:, 0:1] + dgb[...][:, 0:1]
    inv = 1.0 / jnp.maximum(d, 1.0)
    g = d * inv
    sc = jnp.concatenate([s0a[...], s1a[...]], axis=1)
    z = jax.nn.sigmoid(
        jnp.dot(sc, wz[...], preferred_element_type=_f32) * inv + bz[...] * g)
    r = jax.nn.sigmoid(
        jnp.dot(sc, wr[...], preferred_element_type=_f32) * inv + br[...] * g)
    z_out[...] = z
    rh_out[...] = r * hp[...]


_gates = pl.pallas_call(
    _gates_body,
    grid=(NBLK,),
    in_specs=[
        pl.BlockSpec((BLK, D), lambda i: (i, 0)),
        pl.BlockSpec((BLK, D), lambda i: (i + NBLK, 0)),
        pl.BlockSpec((BLK, D), lambda i: (i, 0)),
        pl.BlockSpec((BLK, D), lambda i: (i + NBLK, 0)),
        pl.BlockSpec((BLK, D), lambda i: (i, 0)),
        pl.BlockSpec((2 * D, D), lambda i: (0, 0)),
        pl.BlockSpec((1, D), lambda i: (0, 0)),
        pl.BlockSpec((2 * D, D), lambda i: (0, 0)),
        pl.BlockSpec((1, D), lambda i: (0, 0)),
    ],
    out_specs=[
        pl.BlockSpec((BLK, D), lambda i: (i, 0)),
        pl.BlockSpec((BLK, D), lambda i: (i, 0)),
    ],
    out_shape=[
        jax.ShapeDtypeStruct((N, D), _f32),
        jax.ShapeDtypeStruct((N, D), _f32),
    ],
)


def _out_body(s0a, s2a, s2b, dga, dgb, z, hp, wh, bh, h_out):
    d = dga[...][:, 0:1] + dgb[...][:, 0:1]
    inv = 1.0 / jnp.maximum(d, 1.0)
    g = d * inv
    agg = jnp.concatenate([s0a[...], s2a[...] + s2b[...]], axis=1)
    ht = jnp.tanh(
        jnp.dot(agg, wh[...], preferred_element_type=_f32) * inv + bh[...] * g)
    zz = z[...]
    h_out[...] = zz * hp[...] + (1.0 - zz) * ht


_gru_out = pl.pallas_call(
    _out_body,
    grid=(NBLK,),
    in_specs=[
        pl.BlockSpec((BLK, D), lambda i: (i, 0)),
        pl.BlockSpec((BLK, D), lambda i: (i, 0)),
        pl.BlockSpec((BLK, D), lambda i: (i + NBLK, 0)),
        pl.BlockSpec((BLK, D), lambda i: (i, 0)),
        pl.BlockSpec((BLK, D), lambda i: (i + NBLK, 0)),
        pl.BlockSpec((BLK, D), lambda i: (i, 0)),
        pl.BlockSpec((BLK, D), lambda i: (i, 0)),
        pl.BlockSpec((2 * D, D), lambda i: (0, 0)),
        pl.BlockSpec((1, D), lambda i: (0, 0)),
    ],
    out_specs=pl.BlockSpec((BLK, D), lambda i: (i, 0)),
    out_shape=jax.ShapeDtypeStruct((N, D), _f32),
)


def kernel(X, H_prev, edge_index, Wz, bz, Wr, br, Wh, bh):
    src = edge_index[0]
    dst = edge_index[1]
    xh2 = jnp.concatenate([X, H_prev], axis=0)
    src2 = jnp.concatenate([src, src + N], axis=0)
    zrow = jnp.zeros((K, D), _f32)
    ones = jnp.ones((K, D), _f32)

    s01 = _agg1(xh2, src2, dst, zrow)
    dg = _degk(dst, zrow, ones)
    z, rh = _gates(s01, s01, dg, dg, H_prev,
                   Wz, bz.reshape(1, D), Wr, br.reshape(1, D))
    s2 = _agg2(rh, src, dst, zrow)
    return _gru_out(s01, s2, s2, dg, dg, z, H_prev, Wh, bh.reshape(1, D))


# speedup vs baseline: 1.0527x; 1.0527x over previous
import functools

import jax
import jax.numpy as jnp
from jax import lax
from jax.experimental import pallas as pl
from jax.experimental.pallas import tpu as pltpu
from jax.experimental.pallas import tpu_sc as plsc

N = 10000
E = 320000
D = 128
NC = 2
NS = 16
K = 80
STRIDE = 624
WPT = 640
NCH = WPT // K

_mesh = plsc.VectorSubcoreMesh(core_axis_name="c", subcore_axis_name="s")

_f32 = jnp.float32


def _zero_spmem(zrow_hbm, buf, s_sh, rbase, sem):
    pltpu.sync_copy(zrow_hbm, buf)
    for t in range(NCH):
        pltpu.async_copy(buf, s_sh.at[pl.ds(rbase + t * K, K)], sem)
    for t in range(NCH):
        pltpu.make_async_copy(buf, s_sh.at[pl.ds(rbase, K)], sem).wait()


def _writeout(s_sh, out_ref, obase, rbase, bufs3, semA3, semB3):
    def ld(i, t):
        pltpu.async_copy(s_sh.at[pl.ds(rbase + t * K, K)], bufs3[i],
                         semA3[i])

    def ld_wait(i):
        pltpu.make_async_copy(s_sh.at[pl.ds(rbase, K)], bufs3[i],
                              semA3[i]).wait()

    def st(i, t):
        pltpu.async_copy(bufs3[i], out_ref.at[pl.ds(obase + t * K, K)],
                         semB3[i])

    def st_wait(i):
        pltpu.make_async_copy(bufs3[i], out_ref.at[pl.ds(obase, K)],
                              semB3[i]).wait()

    for t in range(NCH):
        if t >= 3:
            st_wait(t % 3)
        ld(t % 3, t)
        if t >= 1:
            ld_wait((t - 1) % 3)
            st((t - 1) % 3, t - 1)
    ld_wait((NCH - 1) % 3)
    st((NCH - 1) % 3, NCH - 1)
    for c in (NCH - 3, NCH - 2, NCH - 1):
        st_wait(c % 3)


def _gather_scatter_loop(nchunks, ebase, src_off, xref, src_hbm, dst_hbm,
                         s_sh, bufs, semI, semG, semS):
    G = nchunks

    def idx_start(i, g):
        e0 = ebase + g * K
        pltpu.async_copy(src_hbm.at[pl.ds(src_off + e0, K)], bufs[i][0],
                         semI[i][0])
        pltpu.async_copy(dst_hbm.at[pl.ds(e0, K)], bufs[i][1], semI[i][1])

    def idx_wait(i):
        pltpu.make_async_copy(src_hbm.at[pl.ds(0, K)], bufs[i][0],
                              semI[i][0]).wait()
        pltpu.make_async_copy(dst_hbm.at[pl.ds(0, K)], bufs[i][1],
                              semI[i][1]).wait()

    def g_start(i):
        pltpu.async_copy(xref.at[bufs[i][0]], bufs[i][2], semG[i])

    def g_wait(i):
        pltpu.make_async_copy(xref.at[bufs[i][0]], bufs[i][2], semG[i]).wait()

    def s_start(i):
        pltpu.async_copy(bufs[i][2], s_sh.at[bufs[i][1]], semS[i], add=True)

    def s_wait(i):
        pltpu.make_async_copy(bufs[i][2], s_sh.at[bufs[i][1]], semS[i]).wait()

    def sub(i0, t):
        s_wait(i0)
        idx_start(i0, t)
        idx_wait((i0 + 2) % 3)
        g_start((i0 + 2) % 3)
        g_wait((i0 + 1) % 3)
        s_start((i0 + 1) % 3)

    idx_start(0, 0)
    idx_start(1, 1)
    idx_wait(0)
    g_start(0)
    idx_start(2, 2)
    idx_wait(1)
    g_start(1)
    g_wait(0)
    s_start(0)

    nloop = (G - 3) // 3

    def body(g2, carry):
        t = 3 + 3 * g2
        sub(0, t)
        sub(1, t + 1)
        sub(2, t + 2)
        return carry
    lax.fori_loop(0, nloop, body, 0)

    for t in range(3 + 3 * nloop, G):
        sub(t % 3, t)
    i = G % 3
    s_wait(i)
    idx_wait((i + 2) % 3)
    g_start((i + 2) % 3)
    g_wait((i + 1) % 3)
    s_start((i + 1) % 3)
    i = (G + 1) % 3
    s_wait(i)
    g_wait((i + 1) % 3)
    s_start((i + 1) % 3)
    s_wait((G - 1) % 3)


@functools.partial(
    pl.kernel,
    out_type=jax.ShapeDtypeStruct((2 * N, D), _f32),
    mesh=_mesh,
    scratch_types=(
        pltpu.VMEM((K,), jnp.int32),
        pltpu.VMEM((K,), jnp.int32),
        pltpu.VMEM((K, D), _f32),
        pltpu.VMEM((K,), jnp.int32),
        pltpu.VMEM((K,), jnp.int32),
        pltpu.VMEM((K, D), _f32),
        pltpu.VMEM((K,), jnp.int32),
        pltpu.VMEM((K,), jnp.int32),
        pltpu.VMEM((K, D), _f32),
        pltpu.VMEM_SHARED((N, D), _f32),
        pltpu.SemaphoreType.DMA, pltpu.SemaphoreType.DMA,
        pltpu.SemaphoreType.DMA, pltpu.SemaphoreType.DMA,
        pltpu.SemaphoreType.DMA, pltpu.SemaphoreType.DMA,
        pltpu.SemaphoreType.DMA, pltpu.SemaphoreType.DMA,
        pltpu.SemaphoreType.DMA,
        pltpu.SemaphoreType.DMA, pltpu.SemaphoreType.DMA,
        pltpu.SemaphoreType.DMA,
    ),
)
def _agg1(xh2_hbm, src2_hbm, dst_hbm, zrow_hbm,
          s01_out,
          s0, d0, r0_, s1, d1, r1, s2, d2, r2, s_sh,
          i0a, i0b, i1a, i1b, i2a, i2b, g0, g1, g2, t0, t1, t2):
    cid = lax.axis_index("c")
    sid = lax.axis_index("s")
    ept = E // NS
    rbase = sid * STRIDE

    _zero_spmem(zrow_hbm, r0_, s_sh, rbase, g0)
    plsc.subcore_barrier()

    _gather_scatter_loop(ept // K, sid * ept, cid * E, xh2_hbm,
                         src2_hbm, dst_hbm, s_sh,
                         ((s0, d0, r0_), (s1, d1, r1), (s2, d2, r2)),
                         ((i0a, i0b), (i1a, i1b), (i2a, i2b)),
                         (g0, g1, g2), (t0, t1, t2))

    plsc.subcore_barrier()

    _writeout(s_sh, s01_out, cid * N + rbase, rbase,
              (r0_, r1, r2), (g0, g1, g2), (t0, t1, t2))


@functools.partial(
    pl.kernel,
    out_type=jax.ShapeDtypeStruct((2 * N, D), _f32),
    mesh=_mesh,
    scratch_types=(
        pltpu.VMEM((K,), jnp.int32),
        pltpu.VMEM((K,), jnp.int32),
        pltpu.VMEM((K, D), _f32),
        pltpu.VMEM((K, D), _f32),
        pltpu.VMEM((K, D), _f32),
        pltpu.VMEM_SHARED((N, D), _f32),
        pltpu.SemaphoreType.DMA, pltpu.SemaphoreType.DMA,
        pltpu.SemaphoreType.DMA, pltpu.SemaphoreType.DMA,
        pltpu.SemaphoreType.DMA, pltpu.SemaphoreType.DMA,
        pltpu.SemaphoreType.DMA, pltpu.SemaphoreType.DMA,
    ),
)
def _degk(dst_hbm, zrow_hbm, ones_hbm,
          dg_out,
          dstA, dstB, ones_v, r1, r2, d_sh,
          semA, semB, g0, g1, g2, t0, t1, t2):
    cid = lax.axis_index("c")
    sid = lax.axis_index("s")
    epw = E // (NC * NS)
    rbase = sid * STRIDE
    ebase = (cid * NS + sid) * epw
    nchunks = epw // K

    _zero_spmem(zrow_hbm, ones_v, d_sh, rbase, g0)
    pltpu.sync_copy(ones_hbm, ones_v)
    plsc.subcore_barrier()

    def issue(buf_dst, sem, g):
        pltpu.sync_copy(dst_hbm.at[pl.ds(ebase + g * K, K)], buf_dst)
        pltpu.async_copy(ones_v, d_sh.at[buf_dst], sem, add=True)

    def drain(buf_dst, sem):
        pltpu.make_async_copy(ones_v, d_sh.at[buf_dst], sem).wait()

    issue(dstA, semA, 0)
    nloop = (nchunks - 2) // 2

    def body(g2, carry):
        issue(dstB, semB, 2 * g2 + 1)
        drain(dstA, semA)
        issue(dstA, semA, 2 * g2 + 2)
        drain(dstB, semB)
        return carry
    lax.fori_loop(0, nloop, body, 0)

    c0 = 2 * nloop
    if nchunks % 2 == 0:
        issue(dstB, semB, c0 + 1)
        drain(dstA, semA)
        drain(dstB, semB)
    else:
        issue(dstB, semB, c0 + 1)
        drain(dstA, semA)
        issue(dstA, semA, c0 + 2)
        drain(dstB, semB)
        drain(dstA, semA)

    plsc.subcore_barrier()

    _writeout(d_sh, dg_out, cid * N + rbase, rbase,
              (ones_v, r1, r2), (g0, g1, g2), (t0, t1, t2))


@functools.partial(
    pl.kernel,
    out_type=jax.ShapeDtypeStruct((2 * N, D), _f32),
    mesh=_mesh,
    scratch_types=(
        pltpu.VMEM((K,), jnp.int32),
        pltpu.VMEM((K,), jnp.int32),
        pltpu.VMEM((K, D), _f32),
        pltpu.VMEM((K,), jnp.int32),
        pltpu.VMEM((K,), jnp.int32),
        pltpu.VMEM((K, D), _f32),
        pltpu.VMEM((K,), jnp.int32),
        pltpu.VMEM((K,), jnp.int32),
        pltpu.VMEM((K, D), _f32),
        pltpu.VMEM_SHARED((N, D), _f32),
        pltpu.SemaphoreType.DMA, pltpu.SemaphoreType.DMA,
        pltpu.SemaphoreType.DMA, pltpu.SemaphoreType.DMA,
        pltpu.SemaphoreType.DMA, pltpu.SemaphoreType.DMA,
        pltpu.SemaphoreType.DMA, pltpu.SemaphoreType.DMA,
        pltpu.SemaphoreType.DMA,
        pltpu.SemaphoreType.DMA, pltpu.SemaphoreType.DMA,
        pltpu.SemaphoreType.DMA,
    ),
)
def _agg2(rh_hbm, src_hbm, dst_hbm, zrow_hbm,
          s2_out,
          s0, d0, r0_, s1, d1, r1, s2, d2, r2, s_sh,
          i0a, i0b, i1a, i1b, i2a, i2b, g0, g1, g2, t0, t1, t2):
    cid = lax.axis_index("c")
    sid = lax.axis_index("s")
    epw = E // (NC * NS)
    rbase = sid * STRIDE

    _zero_spmem(zrow_hbm, r0_, s_sh, rbase, g0)
    plsc.subcore_barrier()

    wid = cid * NS + sid
    _gather_scatter_loop(epw // K, wid * epw, 0, rh_hbm,
                         src_hbm, dst_hbm, s_sh,
                         ((s0, d0, r0_), (s1, d1, r1), (s2, d2, r2)),
                         ((i0a, i0b), (i1a, i1b), (i2a, i2b)),
                         (g0, g1, g2), (t0, t1, t2))

    plsc.subcore_barrier()

    _writeout(s_sh, s2_out, cid * N + rbase, rbase,
              (r0_, r1, r2), (g0, g1, g2), (t0, t1, t2))


BLK = 2000
NBLK = N // BLK


def _gates_body(s0a, s1a, dga, dgb, hp, wz, bz, wr, br, z_out, rh_out):
    d = dga[...][:, 0:1] + dgb[...][:, 0:1]
    inv = 1.0 / jnp.maximum(d, 1.0)
    g = d * inv
    sc = jnp.concatenate([s0a[...], s1a[...]], axis=1)
    z = jax.nn.sigmoid(
        jnp.dot(sc, wz[...], preferred_element_type=_f32) * inv + bz[...] * g)
    r = jax.nn.sigmoid(
        jnp.dot(sc, wr[...], preferred_element_type=_f32) * inv + br[...] * g)
    z_out[...] = z
    rh_out[...] = r * hp[...]


_gates = pl.pallas_call(
    _gates_body,
    grid=(NBLK,),
    in_specs=[
        pl.BlockSpec((BLK, D), lambda i: (i, 0)),
        pl.BlockSpec((BLK, D), lambda i: (i + NBLK, 0)),
        pl.BlockSpec((BLK, D), lambda i: (i, 0)),
        pl.BlockSpec((BLK, D), lambda i: (i + NBLK, 0)),
        pl.BlockSpec((BLK, D), lambda i: (i, 0)),
        pl.BlockSpec((2 * D, D), lambda i: (0, 0)),
        pl.BlockSpec((1, D), lambda i: (0, 0)),
        pl.BlockSpec((2 * D, D), lambda i: (0, 0)),
        pl.BlockSpec((1, D), lambda i: (0, 0)),
    ],
    out_specs=[
        pl.BlockSpec((BLK, D), lambda i: (i, 0)),
        pl.BlockSpec((BLK, D), lambda i: (i, 0)),
    ],
    out_shape=[
        jax.ShapeDtypeStruct((N, D), _f32),
        jax.ShapeDtypeStruct((N, D), _f32),
    ],
)


def _out_body(s0a, s2a, s2b, dga, dgb, z, hp, wh, bh, h_out):
    d = dga[...][:, 0:1] + dgb[...][:, 0:1]
    inv = 1.0 / jnp.maximum(d, 1.0)
    g = d * inv
    agg = jnp.concatenate([s0a[...], s2a[...] + s2b[...]], axis=1)
    ht = jnp.tanh(
        jnp.dot(agg, wh[...], preferred_element_type=_f32) * inv + bh[...] * g)
    zz = z[...]
    h_out[...] = zz * hp[...] + (1.0 - zz) * ht


_gru_out = pl.pallas_call(
    _out_body,
    grid=(NBLK,),
    in_specs=[
        pl.BlockSpec((BLK, D), lambda i: (i, 0)),
        pl.BlockSpec((BLK, D), lambda i: (i, 0)),
        pl.BlockSpec((BLK, D), lambda i: (i + NBLK, 0)),
        pl.BlockSpec((BLK, D), lambda i: (i, 0)),
        pl.BlockSpec((BLK, D), lambda i: (i + NBLK, 0)),
        pl.BlockSpec((BLK, D), lambda i: (i, 0)),
        pl.BlockSpec((BLK, D), lambda i: (i, 0)),
        pl.BlockSpec((2 * D, D), lambda i: (0, 0)),
        pl.BlockSpec((1, D), lambda i: (0, 0)),
    ],
    out_specs=pl.BlockSpec((BLK, D), lambda i: (i, 0)),
    out_shape=jax.ShapeDtypeStruct((N, D), _f32),
)


def kernel(X, H_prev, edge_index, Wz, bz, Wr, br, Wh, bh):
    src = edge_index[0]
    dst = edge_index[1]
    xh2 = jnp.concatenate([X, H_prev], axis=0)
    src2 = jnp.concatenate([src, src + N], axis=0)
    zrow = jnp.zeros((K, D), _f32)
    ones = jnp.ones((K, D), _f32)

    s01 = _agg1(xh2, src2, dst, zrow)
    dg = _degk(dst, zrow, ones)
    z, rh = _gates(s01, s01, dg, dg, H_prev,
                   Wz, bz.reshape(1, D), Wr, br.reshape(1, D))
    s2 = _agg2(rh, src, dst, zrow)
    return _gru_out(s01, s2, s2, dg, dg, z, H_prev, Wh, bh.reshape(1, D))


# speedup vs baseline: 1.0527x; 1.0001x over previous
import functools

import jax
import jax.numpy as jnp
from jax import lax
from jax.experimental import pallas as pl
from jax.experimental.pallas import tpu as pltpu
from jax.experimental.pallas import tpu_sc as plsc

N = 10000
E = 320000
D = 128
NC = 2
NS = 16
K = 80
STRIDE = 624
WPT = 640
NCH = WPT // K

_mesh = plsc.VectorSubcoreMesh(core_axis_name="c", subcore_axis_name="s")

_f32 = jnp.float32


def _zero_spmem(zrow_hbm, buf, s_sh, rbase, sem):
    pltpu.sync_copy(zrow_hbm, buf)
    for t in range(NCH):
        pltpu.async_copy(buf, s_sh.at[pl.ds(rbase + t * K, K)], sem)
    for t in range(NCH):
        pltpu.make_async_copy(buf, s_sh.at[pl.ds(rbase, K)], sem).wait()


def _writeout(s_sh, out_ref, obase, rbase, bufs3, semA3, semB3):
    def ld(i, t):
        pltpu.async_copy(s_sh.at[pl.ds(rbase + t * K, K)], bufs3[i],
                         semA3[i])

    def ld_wait(i):
        pltpu.make_async_copy(s_sh.at[pl.ds(rbase, K)], bufs3[i],
                              semA3[i]).wait()

    def st(i, t):
        pltpu.async_copy(bufs3[i], out_ref.at[pl.ds(obase + t * K, K)],
                         semB3[i])

    def st_wait(i):
        pltpu.make_async_copy(bufs3[i], out_ref.at[pl.ds(obase, K)],
                              semB3[i]).wait()

    for t in range(NCH):
        if t >= 3:
            st_wait(t % 3)
        ld(t % 3, t)
        if t >= 1:
            ld_wait((t - 1) % 3)
            st((t - 1) % 3, t - 1)
    ld_wait((NCH - 1) % 3)
    st((NCH - 1) % 3, NCH - 1)
    for c in (NCH - 3, NCH - 2, NCH - 1):
        st_wait(c % 3)


def _gather_scatter_loop(nchunks, ebase, src_off, xref, src_hbm, dst_hbm,
                         s_sh, bufs, semI, semG, semS):
    G = nchunks

    def idx_start(i, g):
        e0 = ebase + g * K
        pltpu.async_copy(src_hbm.at[pl.ds(src_off + e0, K)], bufs[i][0],
                         semI[i][0])
        pltpu.async_copy(dst_hbm.at[pl.ds(e0, K)], bufs[i][1], semI[i][1])

    def idx_wait(i):
        pltpu.make_async_copy(src_hbm.at[pl.ds(0, K)], bufs[i][0],
                              semI[i][0]).wait()
        pltpu.make_async_copy(dst_hbm.at[pl.ds(0, K)], bufs[i][1],
                              semI[i][1]).wait()

    def g_start(i):
        pltpu.async_copy(xref.at[bufs[i][0]], bufs[i][2], semG[i])

    def g_wait(i):
        pltpu.make_async_copy(xref.at[bufs[i][0]], bufs[i][2], semG[i]).wait()

    def s_start(i):
        pltpu.async_copy(bufs[i][2], s_sh.at[bufs[i][1]], semS[i], add=True)

    def s_wait(i):
        pltpu.make_async_copy(bufs[i][2], s_sh.at[bufs[i][1]], semS[i]).wait()

    def sub(i0, t):
        s_wait(i0)
        idx_start(i0, t)
        idx_wait((i0 + 2) % 3)
        g_start((i0 + 2) % 3)
        g_wait((i0 + 1) % 3)
        s_start((i0 + 1) % 3)

    idx_start(0, 0)
    idx_start(1, 1)
    idx_wait(0)
    g_start(0)
    idx_start(2, 2)
    idx_wait(1)
    g_start(1)
    g_wait(0)
    s_start(0)

    nloop = (G - 3) // 3

    def body(g2, carry):
        t = 3 + 3 * g2
        sub(0, t)
        sub(1, t + 1)
        sub(2, t + 2)
        return carry
    lax.fori_loop(0, nloop, body, 0)

    for t in range(3 + 3 * nloop, G):
        sub(t % 3, t)
    i = G % 3
    s_wait(i)
    idx_wait((i + 2) % 3)
    g_start((i + 2) % 3)
    g_wait((i + 1) % 3)
    s_start((i + 1) % 3)
    i = (G + 1) % 3
    s_wait(i)
    g_wait((i + 1) % 3)
    s_start((i + 1) % 3)
    s_wait((G - 1) % 3)


@functools.partial(
    pl.kernel,
    out_type=jax.ShapeDtypeStruct((2 * N, D), _f32),
    mesh=_mesh,
    scratch_types=(
        pltpu.VMEM((K,), jnp.int32),
        pltpu.VMEM((K,), jnp.int32),
        pltpu.VMEM((K, D), _f32),
        pltpu.VMEM((K,), jnp.int32),
        pltpu.VMEM((K,), jnp.int32),
        pltpu.VMEM((K, D), _f32),
        pltpu.VMEM((K,), jnp.int32),
        pltpu.VMEM((K,), jnp.int32),
        pltpu.VMEM((K, D), _f32),
        pltpu.VMEM_SHARED((N, D), _f32),
        pltpu.SemaphoreType.DMA, pltpu.SemaphoreType.DMA,
        pltpu.SemaphoreType.DMA, pltpu.SemaphoreType.DMA,
        pltpu.SemaphoreType.DMA, pltpu.SemaphoreType.DMA,
        pltpu.SemaphoreType.DMA, pltpu.SemaphoreType.DMA,
        pltpu.SemaphoreType.DMA,
        pltpu.SemaphoreType.DMA, pltpu.SemaphoreType.DMA,
        pltpu.SemaphoreType.DMA,
    ),
)
def _agg1(xh2_hbm, src2_hbm, dst_hbm, zrow_hbm,
          s01_out,
          s0, d0, r0_, s1, d1, r1, s2, d2, r2, s_sh,
          i0a, i0b, i1a, i1b, i2a, i2b, g0, g1, g2, t0, t1, t2):
    cid = lax.axis_index("c")
    sid = lax.axis_index("s")
    ept = E // NS
    rbase = sid * STRIDE

    _zero_spmem(zrow_hbm, r0_, s_sh, rbase, g0)
    plsc.subcore_barrier()

    _gather_scatter_loop(ept // K, sid * ept, cid * E, xh2_hbm,
                         src2_hbm, dst_hbm, s_sh,
                         ((s0, d0, r0_), (s1, d1, r1), (s2, d2, r2)),
                         ((i0a, i0b), (i1a, i1b), (i2a, i2b)),
                         (g0, g1, g2), (t0, t1, t2))

    plsc.subcore_barrier()

    _writeout(s_sh, s01_out, cid * N + rbase, rbase,
              (r0_, r1, r2), (g0, g1, g2), (t0, t1, t2))


@functools.partial(
    pl.kernel,
    out_type=jax.ShapeDtypeStruct((2 * N, D), _f32),
    mesh=_mesh,
    scratch_types=(
        pltpu.VMEM((K,), jnp.int32),
        pltpu.VMEM((K,), jnp.int32),
        pltpu.VMEM((K, D), _f32),
        pltpu.VMEM((K, D), _f32),
        pltpu.VMEM((K, D), _f32),
        pltpu.VMEM_SHARED((N, D), _f32),
        pltpu.SemaphoreType.DMA, pltpu.SemaphoreType.DMA,
        pltpu.SemaphoreType.DMA, pltpu.SemaphoreType.DMA,
        pltpu.SemaphoreType.DMA, pltpu.SemaphoreType.DMA,
        pltpu.SemaphoreType.DMA, pltpu.SemaphoreType.DMA,
    ),
)
def _degk(dst_hbm, zrow_hbm, ones_hbm,
          dg_out,
          dstA, dstB, ones_v, r1, r2, d_sh,
          semA, semB, g0, g1, g2, t0, t1, t2):
    cid = lax.axis_index("c")
    sid = lax.axis_index("s")
    epw = E // (NC * NS)
    rbase = sid * STRIDE
    ebase = (cid * NS + sid) * epw
    nchunks = epw // K

    _zero_spmem(zrow_hbm, ones_v, d_sh, rbase, g0)
    pltpu.sync_copy(ones_hbm, ones_v)
    plsc.subcore_barrier()

    def issue(buf_dst, sem, g):
        pltpu.sync_copy(dst_hbm.at[pl.ds(ebase + g * K, K)], buf_dst)
        pltpu.async_copy(ones_v, d_sh.at[buf_dst], sem, add=True)

    def drain(buf_dst, sem):
        pltpu.make_async_copy(ones_v, d_sh.at[buf_dst], sem).wait()

    issue(dstA, semA, 0)
    nloop = (nchunks - 2) // 2

    def body(g2, carry):
        issue(dstB, semB, 2 * g2 + 1)
        drain(dstA, semA)
        issue(dstA, semA, 2 * g2 + 2)
        drain(dstB, semB)
        return carry
    lax.fori_loop(0, nloop, body, 0)

    c0 = 2 * nloop
    if nchunks % 2 == 0:
        issue(dstB, semB, c0 + 1)
        drain(dstA, semA)
        drain(dstB, semB)
    else:
        issue(dstB, semB, c0 + 1)
        drain(dstA, semA)
        issue(dstA, semA, c0 + 2)
        drain(dstB, semB)
        drain(dstA, semA)

    plsc.subcore_barrier()

    _writeout(d_sh, dg_out, cid * N + rbase, rbase,
              (ones_v, r1, r2), (g0, g1, g2), (t0, t1, t2))


@functools.partial(
    pl.kernel,
    out_type=jax.ShapeDtypeStruct((2 * N, D), _f32),
    mesh=_mesh,
    scratch_types=(
        pltpu.VMEM((K,), jnp.int32),
        pltpu.VMEM((K,), jnp.int32),
        pltpu.VMEM((K, D), _f32),
        pltpu.VMEM((K,), jnp.int32),
        pltpu.VMEM((K,), jnp.int32),
        pltpu.VMEM((K, D), _f32),
        pltpu.VMEM((K,), jnp.int32),
        pltpu.VMEM((K,), jnp.int32),
        pltpu.VMEM((K, D), _f32),
        pltpu.VMEM_SHARED((N, D), _f32),
        pltpu.SemaphoreType.DMA, pltpu.SemaphoreType.DMA,
        pltpu.SemaphoreType.DMA, pltpu.SemaphoreType.DMA,
        pltpu.SemaphoreType.DMA, pltpu.SemaphoreType.DMA,
        pltpu.SemaphoreType.DMA, pltpu.SemaphoreType.DMA,
        pltpu.SemaphoreType.DMA,
        pltpu.SemaphoreType.DMA, pltpu.SemaphoreType.DMA,
        pltpu.SemaphoreType.DMA,
    ),
)
def _agg2(rh_hbm, src_hbm, dst_hbm, zrow_hbm,
          s2_out,
          s0, d0, r0_, s1, d1, r1, s2, d2, r2, s_sh,
          i0a, i0b, i1a, i1b, i2a, i2b, g0, g1, g2, t0, t1, t2):
    cid = lax.axis_index("c")
    sid = lax.axis_index("s")
    epw = E // (NC * NS)
    rbase = sid * STRIDE

    _zero_spmem(zrow_hbm, r0_, s_sh, rbase, g0)
    plsc.subcore_barrier()

    wid = cid * NS + sid
    _gather_scatter_loop(epw // K, wid * epw, 0, rh_hbm,
                         src_hbm, dst_hbm, s_sh,
                         ((s0, d0, r0_), (s1, d1, r1), (s2, d2, r2)),
                         ((i0a, i0b), (i1a, i1b), (i2a, i2b)),
                         (g0, g1, g2), (t0, t1, t2))

    plsc.subcore_barrier()

    _writeout(s_sh, s2_out, cid * N + rbase, rbase,
              (r0_, r1, r2), (g0, g1, g2), (t0, t1, t2))


BLK = 2000
NBLK = N // BLK


def _gates_body(s0a, s1a, dga, dgb, hp, wz, bz, wr, br, z_out, rh_out):
    d = dga[...][:, 0:1] + dgb[...][:, 0:1]
    inv = 1.0 / jnp.maximum(d, 1.0)
    g = d * inv
    sc = jnp.concatenate([s0a[...], s1a[...]], axis=1)
    z = jax.nn.sigmoid(
        jnp.dot(sc, wz[...], preferred_element_type=_f32) * inv + bz[...] * g)
    r = jax.nn.sigmoid(
        jnp.dot(sc, wr[...], preferred_element_type=_f32) * inv + br[...] * g)
    z_out[...] = z
    rh_out[...] = r * hp[...]


_gates = pl.pallas_call(
    _gates_body,
    grid=(NBLK,),
    in_specs=[
        pl.BlockSpec((BLK, D), lambda i: (i, 0)),
        pl.BlockSpec((BLK, D), lambda i: (i + NBLK, 0)),
        pl.BlockSpec((BLK, D), lambda i: (i, 0)),
        pl.BlockSpec((BLK, D), lambda i: (i + NBLK, 0)),
        pl.BlockSpec((BLK, D), lambda i: (i, 0)),
        pl.BlockSpec((2 * D, D), lambda i: (0, 0)),
        pl.BlockSpec((1, D), lambda i: (0, 0)),
        pl.BlockSpec((2 * D, D), lambda i: (0, 0)),
        pl.BlockSpec((1, D), lambda i: (0, 0)),
    ],
    out_specs=[
        pl.BlockSpec((BLK, D), lambda i: (i, 0)),
        pl.BlockSpec((BLK, D), lambda i: (i, 0)),
    ],
    out_shape=[
        jax.ShapeDtypeStruct((N, D), _f32),
        jax.ShapeDtypeStruct((N, D), _f32),
    ],
)


def _out_body(s0a, s2a, s2b, dga, dgb, z, hp, wh, bh, h_out):
    d = dga[...][:, 0:1] + dgb[...][:, 0:1]
    inv = 1.0 / jnp.maximum(d, 1.0)
    g = d * inv
    agg = jnp.concatenate([s0a[...], s2a[...] + s2b[...]], axis=1)
    ht = jnp.tanh(
        jnp.dot(agg, wh[...], preferred_element_type=_f32) * inv + bh[...] * g)
    zz = z[...]
    h_out[...] = zz * hp[...] + (1.0 - zz) * ht


_gru_out = pl.pallas_call(
    _out_body,
    grid=(NBLK,),
    in_specs=[
        pl.BlockSpec((BLK, D), lambda i: (i, 0)),
        pl.BlockSpec((BLK, D), lambda i: (i, 0)),
        pl.BlockSpec((BLK, D), lambda i: (i + NBLK, 0)),
        pl.BlockSpec((BLK, D), lambda i: (i, 0)),
        pl.BlockSpec((BLK, D), lambda i: (i + NBLK, 0)),
        pl.BlockSpec((BLK, D), lambda i: (i, 0)),
        pl.BlockSpec((BLK, D), lambda i: (i, 0)),
        pl.BlockSpec((2 * D, D), lambda i: (0, 0)),
        pl.BlockSpec((1, D), lambda i: (0, 0)),
    ],
    out_specs=pl.BlockSpec((BLK, D), lambda i: (i, 0)),
    out_shape=jax.ShapeDtypeStruct((N, D), _f32),
)


def kernel(X, H_prev, edge_index, Wz, bz, Wr, br, Wh, bh):
    src = edge_index[0]
    dst = edge_index[1]
    xh2 = jnp.concatenate([X, H_prev], axis=0)
    src2 = jnp.concatenate([src, src + N], axis=0)
    zrow = jnp.zeros((K, D), _f32)
    ones = jnp.ones((K, D), _f32)

    dg = _degk(dst, zrow, ones)
    s01 = _agg1(xh2, src2, dst, zrow)
    z, rh = _gates(s01, s01, dg, dg, H_prev,
                   Wz, bz.reshape(1, D), Wr, br.reshape(1, D))
    s2 = _agg2(rh, src, dst, zrow)
    return _gru_out(s01, s2, s2, dg, dg, z, H_prev, Wh, bh.reshape(1, D))
